# baseline (speedup 1.0000x reference)
import functools

import jax
import jax.numpy as jnp
from jax import lax
from jax.experimental import pallas as pl
from jax.experimental.pallas import tpu as pltpu
from jax.experimental.pallas import tpu_sc as plsc

N = 50000
H = 4
C = 16
HID = H * C
LAT = 256
B = 8
NEG_SLOPE = 0.2

N_PAD = 51200
ROWS_T = N_PAD // 16
BCH = 800
NB = ROWS_T // BCH
E2 = 800000 + N
CH = 128
NCHUNK = 416
NPAIR = NCHUNK // 2
SB = 16
EP = 16 * NCHUNK * CH

RB = 2048
NBLK2 = N_PAD // RB

ROWS = 2000
NBLK = N // ROWS


def _in_body(x_ref, w_ref, b_ref, o_ref):
    o_ref[...] = (jnp.dot(x_ref[...], w_ref[...],
                          preferred_element_type=jnp.float32)
                  + b_ref[...][None, :])


@jax.jit
def _in_proj(x_pad, W_in, b_in):
    return pl.pallas_call(
        _in_body,
        grid=(NBLK2,),
        in_specs=[
            pl.BlockSpec((RB, 8), lambda i: (i, 0)),
            pl.BlockSpec((8, HID), lambda i: (0, 0)),
            pl.BlockSpec((HID,), lambda i: (0,)),
        ],
        out_specs=pl.BlockSpec((RB, HID), lambda i: (i, 0)),
        out_shape=jax.ShapeDtypeStruct((N_PAD, HID), jnp.float32),
    )(x_pad, W_in, b_in)


def _prep_body(h_ref, w_ref, a8_ref, hl_ref, esd_ref, m_ref, macc):
    i = pl.program_id(0)

    @pl.when(i == 0)
    def _init():
        macc[...] = jnp.full_like(macc, -1e30)

    hl = jnp.dot(h_ref[...], w_ref[...], preferred_element_type=jnp.float32)
    esd = jnp.dot(hl, a8_ref[...], preferred_element_type=jnp.float32)
    esd_ref[...] = esd
    for hh in range(H):
        hl_ref[hh] = hl[:, hh * C:(hh + 1) * C]
    macc[...] = jnp.maximum(macc[...], jnp.max(esd, axis=0, keepdims=True))

    @pl.when(i == NBLK2 - 1)
    def _fin():
        m_ref[...] = macc[...]


@jax.jit
def _prep(h, W_g, A8):
    return pl.pallas_call(
        _prep_body,
        grid=(NBLK2,),
        in_specs=[
            pl.BlockSpec((RB, HID), lambda i: (i, 0)),
            pl.BlockSpec((HID, HID), lambda i: (0, 0)),
            pl.BlockSpec((HID, 2 * H), lambda i: (0, 0)),
        ],
        out_specs=[
            pl.BlockSpec((H, RB, C), lambda i: (0, i, 0)),
            pl.BlockSpec((RB, 2 * H), lambda i: (i, 0)),
            pl.BlockSpec((1, 2 * H), lambda i: (0, 0)),
        ],
        out_shape=[
            jax.ShapeDtypeStruct((H, N_PAD, C), jnp.float32),
            jax.ShapeDtypeStruct((N_PAD, 2 * H), jnp.float32),
            jax.ShapeDtypeStruct((1, 2 * H), jnp.float32),
        ],
        scratch_shapes=[pltpu.VMEM((1, 2 * H), jnp.float32)],
    )(h, W_g, A8)


def _edge_body(src_hbm, dst_hbm, hl_hbm, esdT_hbm, m_hbm, zn_hbm, zd_hbm,
               num_hbm, den_hbm,
               m_v, sblk, dblk, dscb, esb, edb, rows, srows, wsc,
               bn, bd, es_sh, ed_sh, acc_num, acc_den,
               stg, sg0, sg1, se0, se1, sd0, sd1, ss0, ss1, sw0, sw1):
    c = lax.axis_index("c")
    s = lax.axis_index("s")
    tile_base = s * (NCHUNK * CH)
    r0 = s * ROWS_T
    gs, esm, dsm, ssm, wsm = (sg0, sg1), (se0, se1), (sd0, sd1), (ss0, ss1), (sw0, sw1)

    for cc in range(2):
      for hh in range(2):
        head = 2 * cc + hh

        @pl.when(c == cc)
        def _core(head=head):
            hl_h = hl_hbm.at[head]
            pltpu.sync_copy(zn_hbm, bn)
            for q in range(NB):
                pltpu.sync_copy(bn, acc_num.at[pl.ds(r0 + q * BCH, BCH)])
            pltpu.sync_copy(zd_hbm, bd)
            for q in range(NB):
                pltpu.sync_copy(bd, acc_den.at[pl.ds(r0 + q * BCH, BCH)])
            for q in range(NB):
                pltpu.sync_copy(esdT_hbm.at[head, pl.ds(r0 + q * BCH, BCH)], bd)
                pltpu.sync_copy(bd, es_sh.at[pl.ds(r0 + q * BCH, BCH)])
            for q in range(NB):
                pltpu.sync_copy(esdT_hbm.at[head + H, pl.ds(r0 + q * BCH, BCH)], bd)
                pltpu.sync_copy(bd, ed_sh.at[pl.ds(r0 + q * BCH, BCH)])
            pltpu.sync_copy(m_hbm.at[head], m_v)
            plsc.subcore_barrier()
            mv = m_v[...]

            def stage_sb(sb):
                b0 = tile_base + sb * (SB * CH)
                c1 = pltpu.async_copy(src_hbm.at[pl.ds(b0, SB * CH)], sblk, stg)
                c2 = pltpu.async_copy(dst_hbm.at[pl.ds(b0, SB * CH)], dblk, stg)
                c1.wait()
                c2.wait()

            def issue_gathers(k, q):
                off = pl.multiple_of(lax.rem(k, SB) * CH, CH)
                pltpu.async_copy(hl_h.at[sblk.at[pl.ds(off, CH)]],
                                 rows.at[q], gs[q])
                pltpu.async_copy(es_sh.at[sblk.at[pl.ds(off, CH)]],
                                 esb.at[q], esm[q])
                pltpu.async_copy(ed_sh.at[dblk.at[pl.ds(off, CH)]],
                                 edb.at[q], dsm[q])

            stage_sb(0)
            issue_gathers(0, 0)

            def pair(m, carry):
                for p in (0, 1):
                    k = 2 * m + p
                    q = 1 - p
                    pltpu.make_async_copy(hl_h.at[sblk.at[pl.ds(0, CH)]],
                                          rows.at[p], gs[p]).wait()
                    pltpu.make_async_copy(es_sh.at[sblk.at[pl.ds(0, CH)]],
                                          esb.at[p], esm[p]).wait()
                    pltpu.make_async_copy(ed_sh.at[dblk.at[pl.ds(0, CH)]],
                                          edb.at[p], dsm[p]).wait()

                    @pl.when(m >= 1)
                    def _drain():
                        pltpu.make_async_copy(zn_hbm.at[pl.ds(0, CH)],
                                              srows.at[p], ssm[p]).wait()
                        pltpu.make_async_copy(zd_hbm.at[pl.ds(0, CH)],
                                              wsc.at[p], wsm[p]).wait()

                    off = pl.multiple_of(lax.rem(k, SB) * CH, CH)
                    for g in range(CH // 16):
                        dscb[p, pl.ds(g * 16, 16)] = dblk[pl.ds(off + g * 16, 16)]

                    def _pf():
                        @pl.when(lax.rem(k + 1, SB) == 0)
                        def _stage():
                            stage_sb((k + 1) // SB)
                        issue_gathers(k + 1, q)
                    if p == 0:
                        _pf()
                    else:
                        pl.when(m < NPAIR - 1)(_pf)

                    for g in range(CH // 16):
                        e = esb[p, pl.ds(g * 16, 16)] + edb[p, pl.ds(g * 16, 16)]
                        e = jnp.maximum(e, NEG_SLOPE * e)
                        w = jnp.exp(e - mv)
                        wsc[p, pl.ds(g * 16, 16)] = w
                        for j in range(16):
                            jj = g * 16 + j
                            srows[p, jj, :] = rows[p, jj, :] * w[j]
                    pltpu.async_copy(srows.at[p], acc_num.at[dscb.at[p]],
                                     ssm[p], add=True)
                    pltpu.async_copy(wsc.at[p], acc_den.at[dscb.at[p]],
                                     wsm[p], add=True)
                return carry

            lax.fori_loop(0, NPAIR, pair, 0)
            for p in (0, 1):
                pltpu.make_async_copy(zn_hbm.at[pl.ds(0, CH)], srows.at[p],
                                      ssm[p]).wait()
                pltpu.make_async_copy(zd_hbm.at[pl.ds(0, CH)], wsc.at[p],
                                      wsm[p]).wait()
            plsc.subcore_barrier()
            for q2 in range(NB):
                pltpu.sync_copy(acc_num.at[pl.ds(r0 + q2 * BCH, BCH)], bn)
                pltpu.sync_copy(bn, num_hbm.at[head, pl.ds(r0 + q2 * BCH, BCH)])
                pltpu.sync_copy(acc_den.at[pl.ds(r0 + q2 * BCH, BCH)], bd)
                pltpu.sync_copy(bd, den_hbm.at[head, pl.ds(r0 + q2 * BCH, BCH)])
            plsc.subcore_barrier()


@jax.jit
def _edge_sc(src2, dst2, hl3, esdT, m_hbm, zn, zd):
    mesh = plsc.VectorSubcoreMesh(core_axis_name="c", subcore_axis_name="s",
                                  num_cores=2, num_subcores=16)
    f = pl.kernel(
        _edge_body,
        out_type=[
            jax.ShapeDtypeStruct((H, N_PAD, C), jnp.float32),
            jax.ShapeDtypeStruct((H, N_PAD), jnp.float32),
        ],
        mesh=mesh,
        compiler_params=pltpu.CompilerParams(needs_layout_passes=False,
                                             use_tc_tiling_on_sc=False),
        scratch_types=[
            pltpu.VMEM((16,), jnp.float32),
            pltpu.VMEM((SB * CH,), jnp.int32),
            pltpu.VMEM((SB * CH,), jnp.int32),
            pltpu.VMEM((2, CH), jnp.int32),
            pltpu.VMEM((2, CH), jnp.float32),
            pltpu.VMEM((2, CH), jnp.float32),
            pltpu.VMEM((2, CH, C), jnp.float32),
            pltpu.VMEM((2, CH, C), jnp.float32),
            pltpu.VMEM((2, CH), jnp.float32),
            pltpu.VMEM((BCH, C), jnp.float32),
            pltpu.VMEM((BCH,), jnp.float32),
            pltpu.VMEM_SHARED((N_PAD,), jnp.float32),
            pltpu.VMEM_SHARED((N_PAD,), jnp.float32),
            pltpu.VMEM_SHARED((N_PAD, C), jnp.float32),
            pltpu.VMEM_SHARED((N_PAD,), jnp.float32),
        ] + [pltpu.SemaphoreType.DMA] * 11,
    )
    return f(src2, dst2, hl3, esdT, m_hbm, zn, zd)


def _fin_body(num_ref, denT_ref, bg_ref, lng_ref, lnb_ref, res_ref, o_ref):
    pieces = []
    for hh in range(H):
        d = denT_ref[:, hh:hh + 1] + 1e-16
        pieces.append(num_ref[hh] / d)
    x = jnp.concatenate(pieces, axis=1) + bg_ref[...][None, :]
    mu = jnp.mean(x, axis=1, keepdims=True)
    var = jnp.mean((x - mu) ** 2, axis=1, keepdims=True)
    y = (x - mu) / jnp.sqrt(var + 1e-5) * lng_ref[...][None, :] \
        + lnb_ref[...][None, :]
    o_ref[...] = jnp.maximum(y, 0.0) + res_ref[...]


@jax.jit
def _finish(num, denT, b_g, ln_g, ln_b, h_res):
    return pl.pallas_call(
        _fin_body,
        grid=(NBLK2,),
        in_specs=[
            pl.BlockSpec((H, RB, C), lambda i: (0, i, 0)),
            pl.BlockSpec((RB, H), lambda i: (i, 0)),
            pl.BlockSpec((HID,), lambda i: (0,)),
            pl.BlockSpec((HID,), lambda i: (0,)),
            pl.BlockSpec((HID,), lambda i: (0,)),
            pl.BlockSpec((RB, HID), lambda i: (i, 0)),
        ],
        out_specs=pl.BlockSpec((RB, HID), lambda i: (i, 0)),
        out_shape=jax.ShapeDtypeStruct((N_PAD, HID), jnp.float32),
    )(num, denT, b_g, ln_g, ln_b, h_res)


def _pool_mlp_body(h_ref, bi_ref, wp1_ref, bp1_ref, wp2_ref, bp2_ref, out_ref,
                   acc_sum, acc_cnt, acc_max):
    i = pl.program_id(0)

    @pl.when(i == 0)
    def _init():
        acc_sum[...] = jnp.zeros_like(acc_sum)
        acc_cnt[...] = jnp.zeros_like(acc_cnt)
        acc_max[...] = jnp.full_like(acc_max, -jnp.inf)

    h = h_ref[...]
    bi = bi_ref[0]
    seg = jax.lax.broadcasted_iota(jnp.int32, (ROWS, B), 1)
    oh = (bi == seg).astype(jnp.float32)
    acc_sum[...] += jax.lax.dot_general(
        oh, h, (((0,), (0,)), ((), ())), preferred_element_type=jnp.float32)
    ones_col = jnp.ones((ROWS, 1), jnp.float32)
    acc_cnt[...] += jax.lax.dot_general(
        oh, ones_col, (((0,), (0,)), ((), ())), preferred_element_type=jnp.float32)
    for b in range(B):
        hm_b = jnp.max(jnp.where(bi == b, h, -jnp.inf), axis=0, keepdims=True)
        acc_max[b:b + 1, :] = jnp.maximum(acc_max[b:b + 1, :], hm_b)

    @pl.when(i == NBLK - 1)
    def _fin():
        mean = acc_sum[...] / jnp.maximum(acc_cnt[...], 1.0)
        g = jnp.concatenate([mean, acc_max[...]], axis=1)
        z = jnp.maximum(jnp.dot(g, wp1_ref[...], preferred_element_type=jnp.float32)
                        + bp1_ref[...][None, :], 0.0)
        out_ref[...] = (jnp.dot(z, wp2_ref[...], preferred_element_type=jnp.float32)
                        + bp2_ref[...][None, :])


@jax.jit
def _pool_mlp(h, batch_idx, W_p1, b_p1, W_p2, b_p2):
    bi3 = batch_idx.astype(jnp.int32).reshape(NBLK, ROWS, 1)
    return pl.pallas_call(
        _pool_mlp_body,
        grid=(NBLK,),
        in_specs=[
            pl.BlockSpec((ROWS, HID), lambda i: (i, 0)),
            pl.BlockSpec((1, ROWS, 1), lambda i: (i, 0, 0)),
            pl.BlockSpec((2 * HID, 2 * LAT), lambda i: (0, 0)),
            pl.BlockSpec((2 * LAT,), lambda i: (0,)),
            pl.BlockSpec((2 * LAT, LAT), lambda i: (0, 0)),
            pl.BlockSpec((LAT,), lambda i: (0,)),
        ],
        out_specs=pl.BlockSpec((B, LAT), lambda i: (0, 0)),
        out_shape=jax.ShapeDtypeStruct((B, LAT), jnp.float32),
        scratch_shapes=[
            pltpu.VMEM((B, HID), jnp.float32),
            pltpu.VMEM((B, 1), jnp.float32),
            pltpu.VMEM((B, HID), jnp.float32),
        ],
    )(h, bi3, W_p1, b_p1, W_p2, b_p2)


def _lrelu(v):
    return jnp.where(v > 0, v, NEG_SLOPE * v)


def kernel(x, edge_index, batch_idx, W_in, b_in, W_g0, a_src0, a_dst0, b_g0, ln_g0, ln_b0, W_g1, a_src1, a_dst1, b_g1, ln_g1, ln_b1, W_g2, a_src2, a_dst2, b_g2, ln_g2, ln_b2, W_p1, b_p1, W_p2, b_p2):
    layers = [(W_g0, a_src0, a_dst0, b_g0, ln_g0, ln_b0),
              (W_g1, a_src1, a_dst1, b_g1, ln_g1, ln_b1),
              (W_g2, a_src2, a_dst2, b_g2, ln_g2, ln_b2)]
    loop = jnp.arange(N, dtype=jnp.int32)
    src2 = jnp.concatenate([edge_index[0].astype(jnp.int32), loop,
                            jnp.full((EP - E2,), N, jnp.int32)])
    dst2 = jnp.concatenate([edge_index[1].astype(jnp.int32), loop,
                            jnp.full((EP - E2,), N, jnp.int32)])
    x_pad = jnp.pad(x.astype(jnp.float32), ((0, N_PAD - N), (0, 2)))
    zn = jnp.zeros((BCH, C), jnp.float32)
    zd = jnp.zeros((BCH,), jnp.float32)
    eye = jnp.eye(H, dtype=jnp.float32)

    h = _in_proj(x_pad, jnp.pad(W_in, ((0, 2), (0, 0))), b_in)
    for (W_g, a_s, a_d, b_g, ln_g, ln_b) in layers:
        A_s = (eye[:, None, :] * a_s[:, :, None]).reshape(HID, H)
        A_d = (eye[:, None, :] * a_d[:, :, None]).reshape(HID, H)
        A8 = jnp.concatenate([A_s, A_d], axis=1)
        hl3, esd, m8 = _prep(h, W_g, A8)
        esdT = esd.T
        M4 = _lrelu(m8[0, :H] + m8[0, H:])
        m_hbm = jnp.broadcast_to(jnp.pad(M4, (0, H))[:, None], (2 * H, 16))
        num, den = _edge_sc(src2, dst2, hl3, esdT, m_hbm, zn, zd)
        denT = den.T
        h = _finish(num, denT, b_g, ln_g, ln_b, h)
    return _pool_mlp(h, batch_idx, W_p1, b_p1, W_p2, b_p2)


# speedup vs baseline: 1.0047x; 1.0047x over previous
import functools

import jax
import jax.numpy as jnp
from jax import lax
from jax.experimental import pallas as pl
from jax.experimental.pallas import tpu as pltpu
from jax.experimental.pallas import tpu_sc as plsc

N = 50000
H = 4
C = 16
HID = H * C
LAT = 256
B = 8
NEG_SLOPE = 0.2

N_PAD = 51200
ROWS_T = N_PAD // 16
BCH = 800
NB = ROWS_T // BCH
E2 = 800000 + N
CH = 128
NCHUNK = 416
NPAIR = NCHUNK // 2
SB = 16
EP = 16 * NCHUNK * CH

RB = 2048
NBLK2 = N_PAD // RB

ROWS = 2000
NBLK = N // ROWS


def _in_body(x_ref, w_ref, b_ref, o_ref):
    o_ref[...] = (jnp.dot(x_ref[...], w_ref[...],
                          preferred_element_type=jnp.float32)
                  + b_ref[...][None, :])


@jax.jit
def _in_proj(x_pad, W_in, b_in):
    return pl.pallas_call(
        _in_body,
        grid=(NBLK2,),
        in_specs=[
            pl.BlockSpec((RB, 8), lambda i: (i, 0)),
            pl.BlockSpec((8, HID), lambda i: (0, 0)),
            pl.BlockSpec((HID,), lambda i: (0,)),
        ],
        out_specs=pl.BlockSpec((RB, HID), lambda i: (i, 0)),
        out_shape=jax.ShapeDtypeStruct((N_PAD, HID), jnp.float32),
    )(x_pad, W_in, b_in)


def _prep_body(h_ref, w_ref, a8_ref, hl_ref, esd_ref, m_ref, macc):
    i = pl.program_id(0)

    @pl.when(i == 0)
    def _init():
        macc[...] = jnp.full_like(macc, -1e30)

    hl = jnp.dot(h_ref[...], w_ref[...], preferred_element_type=jnp.float32)
    esd = jnp.dot(hl, a8_ref[...], preferred_element_type=jnp.float32)
    esd_ref[...] = esd
    for hh in range(H):
        hl_ref[hh] = hl[:, hh * C:(hh + 1) * C]
    macc[...] = jnp.maximum(macc[...], jnp.max(esd, axis=0, keepdims=True))

    @pl.when(i == NBLK2 - 1)
    def _fin():
        m_ref[...] = macc[...]


@jax.jit
def _prep(h, W_g, A8):
    return pl.pallas_call(
        _prep_body,
        grid=(NBLK2,),
        in_specs=[
            pl.BlockSpec((RB, HID), lambda i: (i, 0)),
            pl.BlockSpec((HID, HID), lambda i: (0, 0)),
            pl.BlockSpec((HID, 2 * H), lambda i: (0, 0)),
        ],
        out_specs=[
            pl.BlockSpec((H, RB, C), lambda i: (0, i, 0)),
            pl.BlockSpec((RB, 2 * H), lambda i: (i, 0)),
            pl.BlockSpec((1, 2 * H), lambda i: (0, 0)),
        ],
        out_shape=[
            jax.ShapeDtypeStruct((H, N_PAD, C), jnp.float32),
            jax.ShapeDtypeStruct((N_PAD, 2 * H), jnp.float32),
            jax.ShapeDtypeStruct((1, 2 * H), jnp.float32),
        ],
        scratch_shapes=[pltpu.VMEM((1, 2 * H), jnp.float32)],
    )(h, W_g, A8)


def _edge_body(src_hbm, dst_hbm, hl_hbm, esdT_hbm, m_hbm, zn_hbm, zd_hbm,
               num_hbm, den_hbm,
               m_v, sblk, dblk, dscb, esb, edb, rows, srows, wsc,
               bn, bd, es_sh, ed_sh, acc_num, acc_den,
               stg, sg0, sg1, se0, se1, sd0, sd1, ss0, ss1, sw0, sw1):
    c = lax.axis_index("c")
    s = lax.axis_index("s")
    tile_base = s * (NCHUNK * CH)
    r0 = s * ROWS_T
    gs, esm, dsm, ssm, wsm = (sg0, sg1), (se0, se1), (sd0, sd1), (ss0, ss1), (sw0, sw1)

    for cc in range(2):
      for hh in range(2):
        head = 2 * cc + hh

        @pl.when(c == cc)
        def _core(head=head):
            hl_h = hl_hbm.at[head]
            pltpu.sync_copy(zn_hbm, bn)
            for q in range(NB):
                pltpu.sync_copy(bn, acc_num.at[pl.ds(r0 + q * BCH, BCH)])
            pltpu.sync_copy(zd_hbm, bd)
            for q in range(NB):
                pltpu.sync_copy(bd, acc_den.at[pl.ds(r0 + q * BCH, BCH)])
            for q in range(NB):
                pltpu.sync_copy(esdT_hbm.at[head, pl.ds(r0 + q * BCH, BCH)], bd)
                pltpu.sync_copy(bd, es_sh.at[pl.ds(r0 + q * BCH, BCH)])
            for q in range(NB):
                pltpu.sync_copy(esdT_hbm.at[head + H, pl.ds(r0 + q * BCH, BCH)], bd)
                pltpu.sync_copy(bd, ed_sh.at[pl.ds(r0 + q * BCH, BCH)])
            pltpu.sync_copy(m_hbm.at[head], m_v)
            plsc.subcore_barrier()
            mv = m_v[...]

            def stage_sb(sb):
                b0 = tile_base + sb * (SB * CH)
                c1 = pltpu.async_copy(src_hbm.at[pl.ds(b0, SB * CH)], sblk, stg)
                c2 = pltpu.async_copy(dst_hbm.at[pl.ds(b0, SB * CH)], dblk, stg)
                c1.wait()
                c2.wait()

            def issue_gathers(k, q):
                off = pl.multiple_of(lax.rem(k, SB) * CH, CH)
                pltpu.async_copy(hl_h.at[sblk.at[pl.ds(off, CH)]],
                                 rows.at[q], gs[q])
                pltpu.async_copy(es_sh.at[sblk.at[pl.ds(off, CH)]],
                                 esb.at[q], esm[q])
                pltpu.async_copy(ed_sh.at[dblk.at[pl.ds(off, CH)]],
                                 edb.at[q], dsm[q])

            stage_sb(0)
            issue_gathers(0, 0)

            def pair(m, carry):
                for p in (0, 1):
                    k = 2 * m + p
                    q = 1 - p
                    pltpu.make_async_copy(hl_h.at[sblk.at[pl.ds(0, CH)]],
                                          rows.at[p], gs[p]).wait()
                    pltpu.make_async_copy(es_sh.at[sblk.at[pl.ds(0, CH)]],
                                          esb.at[p], esm[p]).wait()
                    pltpu.make_async_copy(ed_sh.at[dblk.at[pl.ds(0, CH)]],
                                          edb.at[p], dsm[p]).wait()

                    off = pl.multiple_of(lax.rem(k, SB) * CH, CH)
                    for g in range(CH // 16):
                        dscb[p, pl.ds(g * 16, 16)] = dblk[pl.ds(off + g * 16, 16)]

                    def _pf():
                        @pl.when(lax.rem(k + 1, SB) == 0)
                        def _stage():
                            stage_sb((k + 1) // SB)
                        issue_gathers(k + 1, q)
                    if p == 0:
                        _pf()
                    else:
                        pl.when(m < NPAIR - 1)(_pf)

                    for g in range(CH // 16):
                        e = esb[p, pl.ds(g * 16, 16)] + edb[p, pl.ds(g * 16, 16)]
                        e = jnp.maximum(e, NEG_SLOPE * e)
                        w = jnp.exp(e - mv)
                        wsc[p, pl.ds(g * 16, 16)] = w
                        for j in range(16):
                            jj = g * 16 + j
                            srows[p, jj, :] = rows[p, jj, :] * w[j]
                    pltpu.sync_copy(srows.at[p], acc_num.at[dscb.at[p]], add=True)
                    pltpu.sync_copy(wsc.at[p], acc_den.at[dscb.at[p]], add=True)
                return carry

            lax.fori_loop(0, NPAIR, pair, 0)
            plsc.subcore_barrier()
            for q2 in range(NB):
                pltpu.sync_copy(acc_num.at[pl.ds(r0 + q2 * BCH, BCH)], bn)
                pltpu.sync_copy(bn, num_hbm.at[head, pl.ds(r0 + q2 * BCH, BCH)])
                pltpu.sync_copy(acc_den.at[pl.ds(r0 + q2 * BCH, BCH)], bd)
                pltpu.sync_copy(bd, den_hbm.at[head, pl.ds(r0 + q2 * BCH, BCH)])
            plsc.subcore_barrier()


@jax.jit
def _edge_sc(src2, dst2, hl3, esdT, m_hbm, zn, zd):
    mesh = plsc.VectorSubcoreMesh(core_axis_name="c", subcore_axis_name="s",
                                  num_cores=2, num_subcores=16)
    f = pl.kernel(
        _edge_body,
        out_type=[
            jax.ShapeDtypeStruct((H, N_PAD, C), jnp.float32),
            jax.ShapeDtypeStruct((H, N_PAD), jnp.float32),
        ],
        mesh=mesh,
        compiler_params=pltpu.CompilerParams(needs_layout_passes=False,
                                             use_tc_tiling_on_sc=False),
        scratch_types=[
            pltpu.VMEM((16,), jnp.float32),
            pltpu.VMEM((SB * CH,), jnp.int32),
            pltpu.VMEM((SB * CH,), jnp.int32),
            pltpu.VMEM((2, CH), jnp.int32),
            pltpu.VMEM((2, CH), jnp.float32),
            pltpu.VMEM((2, CH), jnp.float32),
            pltpu.VMEM((2, CH, C), jnp.float32),
            pltpu.VMEM((2, CH, C), jnp.float32),
            pltpu.VMEM((2, CH), jnp.float32),
            pltpu.VMEM((BCH, C), jnp.float32),
            pltpu.VMEM((BCH,), jnp.float32),
            pltpu.VMEM_SHARED((N_PAD,), jnp.float32),
            pltpu.VMEM_SHARED((N_PAD,), jnp.float32),
            pltpu.VMEM_SHARED((N_PAD, C), jnp.float32),
            pltpu.VMEM_SHARED((N_PAD,), jnp.float32),
        ] + [pltpu.SemaphoreType.DMA] * 11,
    )
    return f(src2, dst2, hl3, esdT, m_hbm, zn, zd)


def _fin_body(num_ref, denT_ref, bg_ref, lng_ref, lnb_ref, res_ref, o_ref):
    pieces = []
    for hh in range(H):
        d = denT_ref[:, hh:hh + 1] + 1e-16
        pieces.append(num_ref[hh] / d)
    x = jnp.concatenate(pieces, axis=1) + bg_ref[...][None, :]
    mu = jnp.mean(x, axis=1, keepdims=True)
    var = jnp.mean((x - mu) ** 2, axis=1, keepdims=True)
    y = (x - mu) / jnp.sqrt(var + 1e-5) * lng_ref[...][None, :] \
        + lnb_ref[...][None, :]
    o_ref[...] = jnp.maximum(y, 0.0) + res_ref[...]


@jax.jit
def _finish(num, denT, b_g, ln_g, ln_b, h_res):
    return pl.pallas_call(
        _fin_body,
        grid=(NBLK2,),
        in_specs=[
            pl.BlockSpec((H, RB, C), lambda i: (0, i, 0)),
            pl.BlockSpec((RB, H), lambda i: (i, 0)),
            pl.BlockSpec((HID,), lambda i: (0,)),
            pl.BlockSpec((HID,), lambda i: (0,)),
            pl.BlockSpec((HID,), lambda i: (0,)),
            pl.BlockSpec((RB, HID), lambda i: (i, 0)),
        ],
        out_specs=pl.BlockSpec((RB, HID), lambda i: (i, 0)),
        out_shape=jax.ShapeDtypeStruct((N_PAD, HID), jnp.float32),
    )(num, denT, b_g, ln_g, ln_b, h_res)


def _pool_mlp_body(h_ref, bi_ref, wp1_ref, bp1_ref, wp2_ref, bp2_ref, out_ref,
                   acc_sum, acc_cnt, acc_max):
    i = pl.program_id(0)

    @pl.when(i == 0)
    def _init():
        acc_sum[...] = jnp.zeros_like(acc_sum)
        acc_cnt[...] = jnp.zeros_like(acc_cnt)
        acc_max[...] = jnp.full_like(acc_max, -jnp.inf)

    h = h_ref[...]
    bi = bi_ref[0]
    seg = jax.lax.broadcasted_iota(jnp.int32, (ROWS, B), 1)
    oh = (bi == seg).astype(jnp.float32)
    acc_sum[...] += jax.lax.dot_general(
        oh, h, (((0,), (0,)), ((), ())), preferred_element_type=jnp.float32)
    ones_col = jnp.ones((ROWS, 1), jnp.float32)
    acc_cnt[...] += jax.lax.dot_general(
        oh, ones_col, (((0,), (0,)), ((), ())), preferred_element_type=jnp.float32)
    for b in range(B):
        hm_b = jnp.max(jnp.where(bi == b, h, -jnp.inf), axis=0, keepdims=True)
        acc_max[b:b + 1, :] = jnp.maximum(acc_max[b:b + 1, :], hm_b)

    @pl.when(i == NBLK - 1)
    def _fin():
        mean = acc_sum[...] / jnp.maximum(acc_cnt[...], 1.0)
        g = jnp.concatenate([mean, acc_max[...]], axis=1)
        z = jnp.maximum(jnp.dot(g, wp1_ref[...], preferred_element_type=jnp.float32)
                        + bp1_ref[...][None, :], 0.0)
        out_ref[...] = (jnp.dot(z, wp2_ref[...], preferred_element_type=jnp.float32)
                        + bp2_ref[...][None, :])


@jax.jit
def _pool_mlp(h, batch_idx, W_p1, b_p1, W_p2, b_p2):
    bi3 = batch_idx.astype(jnp.int32).reshape(NBLK, ROWS, 1)
    return pl.pallas_call(
        _pool_mlp_body,
        grid=(NBLK,),
        in_specs=[
            pl.BlockSpec((ROWS, HID), lambda i: (i, 0)),
            pl.BlockSpec((1, ROWS, 1), lambda i: (i, 0, 0)),
            pl.BlockSpec((2 * HID, 2 * LAT), lambda i: (0, 0)),
            pl.BlockSpec((2 * LAT,), lambda i: (0,)),
            pl.BlockSpec((2 * LAT, LAT), lambda i: (0, 0)),
            pl.BlockSpec((LAT,), lambda i: (0,)),
        ],
        out_specs=pl.BlockSpec((B, LAT), lambda i: (0, 0)),
        out_shape=jax.ShapeDtypeStruct((B, LAT), jnp.float32),
        scratch_shapes=[
            pltpu.VMEM((B, HID), jnp.float32),
            pltpu.VMEM((B, 1), jnp.float32),
            pltpu.VMEM((B, HID), jnp.float32),
        ],
    )(h, bi3, W_p1, b_p1, W_p2, b_p2)


def _lrelu(v):
    return jnp.where(v > 0, v, NEG_SLOPE * v)


def kernel(x, edge_index, batch_idx, W_in, b_in, W_g0, a_src0, a_dst0, b_g0, ln_g0, ln_b0, W_g1, a_src1, a_dst1, b_g1, ln_g1, ln_b1, W_g2, a_src2, a_dst2, b_g2, ln_g2, ln_b2, W_p1, b_p1, W_p2, b_p2):
    layers = [(W_g0, a_src0, a_dst0, b_g0, ln_g0, ln_b0),
              (W_g1, a_src1, a_dst1, b_g1, ln_g1, ln_b1),
              (W_g2, a_src2, a_dst2, b_g2, ln_g2, ln_b2)]
    loop = jnp.arange(N, dtype=jnp.int32)
    src2 = jnp.concatenate([edge_index[0].astype(jnp.int32), loop,
                            jnp.full((EP - E2,), N, jnp.int32)])
    dst2 = jnp.concatenate([edge_index[1].astype(jnp.int32), loop,
                            jnp.full((EP - E2,), N, jnp.int32)])
    x_pad = jnp.pad(x.astype(jnp.float32), ((0, N_PAD - N), (0, 2)))
    zn = jnp.zeros((BCH, C), jnp.float32)
    zd = jnp.zeros((BCH,), jnp.float32)
    eye = jnp.eye(H, dtype=jnp.float32)

    h = _in_proj(x_pad, jnp.pad(W_in, ((0, 2), (0, 0))), b_in)
    for (W_g, a_s, a_d, b_g, ln_g, ln_b) in layers:
        A_s = (eye[:, None, :] * a_s[:, :, None]).reshape(HID, H)
        A_d = (eye[:, None, :] * a_d[:, :, None]).reshape(HID, H)
        A8 = jnp.concatenate([A_s, A_d], axis=1)
        hl3, esd, m8 = _prep(h, W_g, A8)
        esdT = esd.T
        M4 = _lrelu(m8[0, :H] + m8[0, H:])
        m_hbm = jnp.broadcast_to(jnp.pad(M4, (0, H))[:, None], (2 * H, 16))
        num, den = _edge_sc(src2, dst2, hl3, esdT, m_hbm, zn, zd)
        denT = den.T
        h = _finish(num, denT, b_g, ln_g, ln_b, h)
    return _pool_mlp(h, batch_idx, W_p1, b_p1, W_p2, b_p2)
